# Initial kernel scaffold; baseline (speedup 1.0000x reference)
import jax
import jax.numpy as jnp
from jax.experimental import pallas as pl


def kernel(Q, S, R, Pos):
    raise NotImplementedError("write your pallas kernel here")


# speedup vs baseline: 13.4152x; 13.4152x over previous
import functools

import jax
import jax.numpy as jnp
from jax import lax
from jax.experimental import pallas as pl
from jax.experimental.pallas import tpu as pltpu
from jax.experimental.pallas import tpu_sc as plsc


def _scores_tc(qmat, sflat, bq=392):
    B, hw, C = qmat.shape
    HW = sflat.shape[2]

    def body(q_ref, s_ref, a_ref):
        a_ref[0] = lax.dot_general(
            q_ref[0], s_ref[0], (((1,), (0,)), ((), ())),
            precision=lax.Precision.HIGHEST,
            preferred_element_type=jnp.float32)

    return pl.pallas_call(
        body,
        grid=(B, hw // bq),
        in_specs=[
            pl.BlockSpec((1, bq, C), lambda b, i: (b, i, 0)),
            pl.BlockSpec((1, C, HW), lambda b, i: (b, 0, 0)),
        ],
        out_specs=pl.BlockSpec((1, bq, HW), lambda b, i: (b, i, 0)),
        out_shape=jax.ShapeDtypeStruct((B, hw, HW), jnp.float32),
    )(qmat, sflat)


_CQ = 4


def _lane_reduce(v, op):
    vals = [v[i] for i in range(16)]
    while len(vals) > 1:
        vals = [op(vals[i], vals[i + 1]) for i in range(0, len(vals) - 1, 2)] + (
            [vals[-1]] if len(vals) % 2 else [])
    return vals[0]


def _attend_sc(a_flat, rt, px, py, W, HW, C):
    NQ, K = px.shape
    CP = rt.shape[1]
    NW = 32
    QPW = NQ // NW
    NCH = QPW // _CQ
    NC16 = C // 16

    mesh = plsc.VectorSubcoreMesh(core_axis_name="c", subcore_axis_name="s")

    @functools.partial(
        pl.kernel,
        out_type=[jax.ShapeDtypeStruct((NQ, K), jnp.float32),
                  jax.ShapeDtypeStruct((NQ, C), jnp.float32)],
        mesh=mesh,
        scratch_types=[
            pltpu.VMEM((_CQ, K), jnp.int32),
            pltpu.VMEM((_CQ, K), jnp.int32),
            pltpu.VMEM((_CQ * K,), jnp.int32),
            pltpu.VMEM((_CQ * K,), jnp.int32),
            pltpu.VMEM((_CQ * K,), jnp.float32),
            pltpu.VMEM((_CQ, K), jnp.float32),
            pltpu.VMEM((_CQ * K, CP), jnp.float32),
            pltpu.VMEM((_CQ, C), jnp.float32),
            pltpu.SemaphoreType.DMA,
            pltpu.SemaphoreType.DMA,
        ],
    )
    def attend(a_hbm, rt_hbm, px_hbm, py_hbm, w_hbm, o_hbm,
               px_v, py_v, sidx_v, vidx_v, sc_v, w_v, vr_v, o_v, sem1, sem2):
        cid = lax.axis_index("c")
        sid = lax.axis_index("s")
        wid = sid * 2 + cid
        qbase = wid * QPW
        vbase = (wid // 16) * HW

        def chunk(t, carry):
            q0 = qbase + t * _CQ
            pltpu.sync_copy(px_hbm.at[pl.ds(q0, _CQ)], px_v)
            pltpu.sync_copy(py_hbm.at[pl.ds(q0, _CQ)], py_v)
            for r in range(_CQ):
                for j in range(K // 16):
                    pf = px_v[r, pl.ds(j * 16, 16)] * W + py_v[r, pl.ds(j * 16, 16)]
                    sidx_v[pl.ds(r * K + j * 16, 16)] = (q0 + r) * HW + pf
                    vidx_v[pl.ds(r * K + j * 16, 16)] = vbase + pf
            pltpu.async_copy(a_hbm.at[sidx_v], sc_v, sem1).wait()
            vcp = pltpu.async_copy(rt_hbm.at[vidx_v], vr_v, sem2)
            for r in range(_CQ):
                s0 = sc_v[pl.ds(r * K, 16)]
                s1 = sc_v[pl.ds(r * K + 16, 16)]
                vm = jnp.maximum(s0, s1)
                m = _lane_reduce(vm, jnp.maximum)
                e0 = jnp.exp(s0 - m)
                e1 = jnp.exp(s1 - m)
                den = _lane_reduce(e0 + e1, lax.add)
                w_v[r, pl.ds(0, 16)] = e0 / den
                w_v[r, pl.ds(16, 16)] = e1 / den
            pltpu.sync_copy(w_v, w_hbm.at[pl.ds(q0, _CQ)])
            vcp.wait()
            for r in range(_CQ):
                acc = [jnp.zeros((16,), jnp.float32) for _ in range(NC16)]
                for j in range(K // 16):
                    wv = w_v[r, pl.ds(j * 16, 16)]
                    for ii in range(16):
                        ws = wv[ii]
                        row = r * K + j * 16 + ii
                        for cc in range(NC16):
                            acc[cc] = acc[cc] + ws * vr_v[row, pl.ds(cc * 16, 16)]
                for cc in range(NC16):
                    o_v[r, pl.ds(cc * 16, 16)] = acc[cc]
            pltpu.sync_copy(o_v, o_hbm.at[pl.ds(q0, _CQ)])
            return carry

        lax.fori_loop(0, NCH, chunk, 0)

    return attend(a_flat, rt, px, py)


def kernel(Q, S, R, Pos):
    B, C, h, w = Q.shape
    H, W = S.shape[2], S.shape[3]
    HW = H * W
    hw = h * w
    K = Pos.shape[-1]

    qmat = Q.reshape(B, C, hw).transpose(0, 2, 1)
    sflat = S.reshape(B, C, HW)
    a = _scores_tc(qmat, sflat)

    rt = R.reshape(B, C, HW).transpose(0, 2, 1).reshape(B * HW, C)
    CP = 256
    rt = jnp.pad(rt, ((0, 0), (0, CP - C)))
    px = Pos[:, 0].reshape(B * hw, K)
    py = Pos[:, 1].reshape(B * hw, K)

    wts, orows = _attend_sc(a.reshape(B * hw * HW), rt, px, py, W, HW, C)

    M = wts.reshape(B, hw, 1, K)
    buf = orows.reshape(B, h, w, C).transpose(0, 3, 1, 2)
    return (buf, M)


# speedup vs baseline: 19.9093x; 1.4841x over previous
import functools

import jax
import jax.numpy as jnp
from jax import lax
from jax.experimental import pallas as pl
from jax.experimental.pallas import tpu as pltpu
from jax.experimental.pallas import tpu_sc as plsc


def _scores_tc(qmat, sflat, bq=392):
    B, hw, C = qmat.shape
    HW = sflat.shape[2]

    def body(q_ref, s_ref, a_ref):
        a_ref[0] = lax.dot_general(
            q_ref[0], s_ref[0], (((1,), (0,)), ((), ())),
            precision=lax.Precision.HIGHEST,
            preferred_element_type=jnp.float32)

    return pl.pallas_call(
        body,
        grid=(B, hw // bq),
        in_specs=[
            pl.BlockSpec((1, bq, C), lambda b, i: (b, i, 0)),
            pl.BlockSpec((1, C, HW), lambda b, i: (b, 0, 0)),
        ],
        out_specs=pl.BlockSpec((1, bq, HW), lambda b, i: (b, i, 0)),
        out_shape=jax.ShapeDtypeStruct((B, hw, HW), jnp.float32),
    )(qmat, sflat)


_CQ = 2
_IQ = 4


def _lane_reduce(v, op):
    vals = [v[i] for i in range(16)]
    while len(vals) > 1:
        vals = [op(vals[i], vals[i + 1]) for i in range(0, len(vals) - 1, 2)] + (
            [vals[-1]] if len(vals) % 2 else [])
    return vals[0]


def _attend_sc(a_flat, rt, px, py, W, HW, C, NQ, K):
    CP = rt.shape[1]
    NW = 32
    QPW = NQ // NW
    NCH = QPW // _CQ
    NC16 = C // 16

    NIC = QPW // _IQ
    NVC = QPW // _CQ
    mesh = plsc.VectorSubcoreMesh(core_axis_name="c", subcore_axis_name="s")

    @functools.partial(
        pl.kernel,
        out_type=[jax.ShapeDtypeStruct((NQ * K,), jnp.float32),
                  jax.ShapeDtypeStruct((NQ * C,), jnp.float32)],
        mesh=mesh,
        scratch_types=[
            pltpu.VMEM((QPW * K,), jnp.int32),
            pltpu.VMEM((QPW * K,), jnp.int32),
            pltpu.VMEM((QPW * K,), jnp.int32),
            pltpu.VMEM((QPW * K,), jnp.int32),
            pltpu.VMEM((QPW * K,), jnp.float32),
            pltpu.VMEM((QPW * K,), jnp.float32),
            pltpu.VMEM((_CQ * K, CP), jnp.float32),
            pltpu.VMEM((_CQ * K, CP), jnp.float32),
            pltpu.VMEM((QPW * C,), jnp.float32),
            pltpu.SemaphoreType.DMA,
            pltpu.SemaphoreType.DMA,
            pltpu.SemaphoreType.DMA,
        ],
    )
    def attend(a_hbm, rt_hbm, px_hbm, py_hbm, w_hbm, o_hbm,
               pxa, pya, sidx, vidx, sca, wa, vr0, vr1, oa,
               sem_s, sem_v0, sem_v1):
        cid = lax.axis_index("c")
        sid = lax.axis_index("s")
        wid = sid * 2 + cid
        qbase = wid * QPW
        vbase = (wid // 16) * HW
        SB = _IQ * K
        VB = _CQ * K

        pltpu.sync_copy(px_hbm.at[pl.ds(qbase * K, QPW * K)], pxa)
        pltpu.sync_copy(py_hbm.at[pl.ds(qbase * K, QPW * K)], pya)

        def build(t, carry):
            for r in range(_IQ):
                q = t * _IQ + r
                for j in range(K // 16):
                    o = q * K + j * 16
                    pf = pxa[pl.ds(o, 16)] * W + pya[pl.ds(o, 16)]
                    sidx[pl.ds(o, 16)] = (qbase + q) * HW + pf
                    vidx[pl.ds(o, 16)] = vbase + pf
            pltpu.async_copy(a_hbm.at[sidx.at[pl.ds(t * SB, SB)]],
                             sca.at[pl.ds(t * SB, SB)], sem_s)
            return carry

        lax.fori_loop(0, NIC, build, 0)

        def drain_s(t, carry):
            pltpu.make_async_copy(a_hbm.at[sidx.at[pl.ds(t * SB, SB)]],
                                  sca.at[pl.ds(t * SB, SB)], sem_s).wait()
            return carry

        lax.fori_loop(0, NIC, drain_s, 0)

        def smax(t, carry):
            for r in range(_IQ):
                q = t * _IQ + r
                s0 = sca[pl.ds(q * K, 16)]
                s1 = sca[pl.ds(q * K + 16, 16)]
                vm = jnp.maximum(s0, s1)
                m = _lane_reduce(vm, jnp.maximum)
                e0 = jnp.exp(s0 - m)
                e1 = jnp.exp(s1 - m)
                den = _lane_reduce(e0 + e1, lax.add)
                wa[pl.ds(q * K, 16)] = e0 / den
                wa[pl.ds(q * K + 16, 16)] = e1 / den
            return carry

        lax.fori_loop(0, NIC, smax, 0)

        bufs = (vr0, vr1)
        sems = (sem_v0, sem_v1)
        pltpu.async_copy(rt_hbm.at[vidx.at[pl.ds(0, VB)]], vr0, sem_v0)

        def pipe(tt, carry):
            for half in range(2):
                t = tt * 2 + half
                nxt = t + 1
                if half == 0:
                    pltpu.async_copy(rt_hbm.at[vidx.at[pl.ds(nxt * VB, VB)]],
                                     bufs[1], sems[1])
                else:
                    @pl.when(nxt < NVC)
                    def _():
                        pltpu.async_copy(rt_hbm.at[vidx.at[pl.ds(nxt * VB, VB)]],
                                         bufs[0], sems[0])
                vr = bufs[half]
                pltpu.make_async_copy(rt_hbm.at[vidx.at[pl.ds(t * VB, VB)]],
                                      vr, sems[half]).wait()
                for rr in range(_CQ):
                    q = t * _CQ + rr
                    acc = [jnp.zeros((16,), jnp.float32) for _ in range(NC16)]
                    for j in range(K // 16):
                        wv = wa[pl.ds(q * K + j * 16, 16)]
                        for ii in range(16):
                            ws = wv[ii]
                            row = rr * K + j * 16 + ii
                            for cc in range(NC16):
                                acc[cc] = acc[cc] + ws * vr[row, pl.ds(cc * 16, 16)]
                    for cc in range(NC16):
                        oa[pl.ds(q * C + cc * 16, 16)] = acc[cc]
            return carry

        lax.fori_loop(0, NVC // 2, pipe, 0)

        pltpu.sync_copy(wa, w_hbm.at[pl.ds(qbase * K, QPW * K)])
        pltpu.sync_copy(oa, o_hbm.at[pl.ds(qbase * C, QPW * C)])

    return attend(a_flat, rt, px, py)


def kernel(Q, S, R, Pos):
    B, C, h, w = Q.shape
    H, W = S.shape[2], S.shape[3]
    HW = H * W
    hw = h * w
    K = Pos.shape[-1]

    qmat = Q.reshape(B, C, hw).transpose(0, 2, 1)
    sflat = S.reshape(B, C, HW)
    a = _scores_tc(qmat, sflat)

    rt = R.reshape(B, C, HW).transpose(0, 2, 1).reshape(B * HW, C)
    CP = 256
    rt = jnp.pad(rt, ((0, 0), (0, CP - C)))
    px = Pos[:, 0].reshape(B * hw * K)
    py = Pos[:, 1].reshape(B * hw * K)

    wts, orows = _attend_sc(a.reshape(B * hw * HW), rt, px, py,
                            W, HW, C, B * hw, K)

    M = wts.reshape(B, hw, 1, K)
    buf = orows.reshape(B, h, w, C).transpose(0, 3, 1, 2)
    return (buf, M)


# speedup vs baseline: 22.6122x; 1.1358x over previous
import functools

import jax
import jax.numpy as jnp
from jax import lax
from jax.experimental import pallas as pl
from jax.experimental.pallas import tpu as pltpu
from jax.experimental.pallas import tpu_sc as plsc


def _scores_tc(qh, ql, sh, sl, bq=392):
    hw, C = qh.shape
    HW = sh.shape[1]
    nbq = hw // bq

    def body(qh_ref, ql_ref, sh_ref, sl_ref, a_ref):
        dn = (((1,), (0,)), ((), ()))

        def dot(x, y):
            return lax.dot_general(x, y, dn,
                                   preferred_element_type=jnp.float32)

        a_ref[...] = dot(qh_ref[...], sh_ref[...]) + (
            dot(qh_ref[...], sl_ref[...]) + dot(ql_ref[...], sh_ref[...]))

    qspec = pl.BlockSpec((bq, C), lambda i: (i, 0))
    sspec = pl.BlockSpec((C, HW), lambda i: (0, 0))
    return pl.pallas_call(
        body,
        grid=(nbq,),
        in_specs=[qspec, qspec, sspec, sspec],
        out_specs=pl.BlockSpec((bq, HW), lambda i: (i, 0)),
        out_shape=jax.ShapeDtypeStruct((hw, HW), jnp.float32),
    )(qh, ql, sh, sl)


_CQ = 2


def _lane_reduce(v, op):
    vals = [v[i] for i in range(16)]
    while len(vals) > 1:
        vals = [op(vals[i], vals[i + 1]) for i in range(0, len(vals) - 1, 2)] + (
            [vals[-1]] if len(vals) % 2 else [])
    return vals[0]


def _attend_sc(a_flat, rt, px, py, b, W, HW, C, K):
    CP = rt.shape[1]
    NQ = px.shape[0] // K
    NW = 32
    QPW = NQ // NW
    NC16 = C // 16
    NCH = QPW // _CQ
    NPAIR = NCH // 2
    mesh = plsc.VectorSubcoreMesh(core_axis_name="c", subcore_axis_name="s")

    @functools.partial(
        pl.kernel,
        out_type=[jax.ShapeDtypeStruct((NQ * K,), jnp.float32),
                  jax.ShapeDtypeStruct((NQ * C,), jnp.float32)],
        mesh=mesh,
        scratch_types=[
            pltpu.VMEM((QPW * K,), jnp.int32),
            pltpu.VMEM((QPW * K,), jnp.int32),
            pltpu.VMEM((QPW * K,), jnp.int32),
            pltpu.VMEM((QPW * K,), jnp.int32),
            pltpu.VMEM((QPW * K,), jnp.float32),
            pltpu.VMEM((QPW * K,), jnp.float32),
            pltpu.VMEM((_CQ * K, CP), jnp.float32),
            pltpu.VMEM((_CQ * K, CP), jnp.float32),
            pltpu.VMEM((QPW * C,), jnp.float32),
            pltpu.SemaphoreType.DMA,
            pltpu.SemaphoreType.DMA,
            pltpu.SemaphoreType.DMA,
        ],
    )
    def attend(a_hbm, rt_hbm, px_hbm, py_hbm, w_hbm, o_hbm,
               pxa, pya, sidx, vidx, sca, wa, vr0, vr1, oa,
               sem_s, sem_v0, sem_v1):
        cid = lax.axis_index("c")
        sid = lax.axis_index("s")
        wid = sid * 2 + cid
        qbase = wid * QPW
        vbase = b * HW
        SB = _CQ * K

        pltpu.sync_copy(px_hbm.at[pl.ds(qbase * K, QPW * K)], pxa)
        pltpu.sync_copy(py_hbm.at[pl.ds(qbase * K, QPW * K)], pya)

        def s_copy(t):
            return pltpu.make_async_copy(
                a_hbm.at[sidx.at[pl.ds(t * SB, SB)]],
                sca.at[pl.ds(t * SB, SB)], sem_s)

        vrow = (vr0, vr1)
        vsem = (sem_v0, sem_v1)

        def v_copy(t, buf):
            return pltpu.make_async_copy(
                rt_hbm.at[vidx.at[pl.ds(t * SB, SB)]], vrow[buf], vsem[buf])

        def build(t, carry):
            for r in range(_CQ):
                q = t * _CQ + r
                for j in range(K // 16):
                    o = q * K + j * 16
                    pf = pxa[pl.ds(o, 16)] * W + pya[pl.ds(o, 16)]
                    sidx[pl.ds(o, 16)] = (qbase + q) * HW + pf
                    vidx[pl.ds(o, 16)] = vbase + pf
            s_copy(t).start()
            return carry

        lax.fori_loop(0, NCH, build, 0)

        def drain_s(t, carry):
            s_copy(t).wait()
            return carry

        lax.fori_loop(0, NCH, drain_s, 0)

        def smax(t, carry):
            for r in range(_CQ):
                q = t * _CQ + r
                s0 = sca[pl.ds(q * K, 16)]
                s1 = sca[pl.ds(q * K + 16, 16)]
                vm = jnp.maximum(s0, s1)
                m = _lane_reduce(vm, jnp.maximum)
                e0 = jnp.exp(s0 - m)
                e1 = jnp.exp(s1 - m)
                den = _lane_reduce(e0 + e1, lax.add)
                wa[pl.ds(q * K, 16)] = e0 / den
                wa[pl.ds(q * K + 16, 16)] = e1 / den
            return carry

        lax.fori_loop(0, NCH, smax, 0)

        def combine(t, vr):
            for rr in range(_CQ):
                q = t * _CQ + rr
                acc = [jnp.zeros((16,), jnp.float32) for _ in range(NC16)]
                for j in range(K // 16):
                    wv = wa[pl.ds(q * K + j * 16, 16)]
                    for ii in range(16):
                        ws = wv[ii]
                        row = rr * K + j * 16 + ii
                        for cc in range(NC16):
                            acc[cc] = acc[cc] + ws * vr[row, pl.ds(cc * 16, 16)]
                for cc in range(NC16):
                    oa[pl.ds(q * C + cc * 16, 16)] = acc[cc]

        v_copy(0, 0).start()

        def pipe(tt, carry):
            for half in range(2):
                t = tt * 2 + half
                v_copy(t + 1, 1 - half).start()
                v_copy(t, half).wait()
                combine(t, vrow[half])
            return carry

        lax.fori_loop(0, NPAIR, pipe, 0)
        v_copy(NCH - 1, 0).wait()
        combine(NCH - 1, vrow[0])

        pltpu.sync_copy(wa, w_hbm.at[pl.ds(qbase * K, QPW * K)])
        pltpu.sync_copy(oa, o_hbm.at[pl.ds(qbase * C, QPW * C)])

    return attend(a_flat, rt, px, py)


def kernel(Q, S, R, Pos):
    B, C, h, w = Q.shape
    H, W = S.shape[2], S.shape[3]
    HW = H * W
    hw = h * w
    K = Pos.shape[-1]

    qmat = Q.reshape(B, C, hw).transpose(0, 2, 1)
    qh = qmat.astype(jnp.bfloat16)
    ql = (qmat - qh.astype(jnp.float32)).astype(jnp.bfloat16)
    sflat = S.reshape(B, C, HW)
    sh = sflat.astype(jnp.bfloat16)
    sl = (sflat - sh.astype(jnp.float32)).astype(jnp.bfloat16)

    rt = R.reshape(B, C, HW).transpose(0, 2, 1).reshape(B * HW, C)
    CP = 256
    rt = jnp.pad(rt, ((0, 0), (0, CP - C)))
    pxb = Pos[:, 0].reshape(B, hw * K)
    pyb = Pos[:, 1].reshape(B, hw * K)

    wts, orows = [], []
    for b in range(B):
        a = _scores_tc(qh[b], ql[b], sh[b], sl[b])
        wb, ob = _attend_sc(a.reshape(hw * HW), rt, pxb[b], pyb[b],
                            b, W, HW, C, K)
        wts.append(wb.reshape(hw, 1, K))
        orows.append(ob.reshape(h, w, C))

    M = jnp.stack(wts)
    buf = jnp.stack(orows).transpose(0, 3, 1, 2)
    return (buf, M)


# speedup vs baseline: 23.2429x; 1.0279x over previous
import functools

import jax
import jax.numpy as jnp
from jax import lax
from jax.experimental import pallas as pl
from jax.experimental.pallas import tpu as pltpu
from jax.experimental.pallas import tpu_sc as plsc


def _scores_tc(qh, ql, sh, sl, bq=392):
    hw, C = qh.shape
    HW = sh.shape[1]
    nbq = hw // bq

    def body(qh_ref, ql_ref, sh_ref, sl_ref, a_ref):
        dn = (((1,), (0,)), ((), ()))

        def dot(x, y):
            return lax.dot_general(x, y, dn,
                                   preferred_element_type=jnp.float32)

        a_ref[...] = dot(qh_ref[...], sh_ref[...]) + (
            dot(qh_ref[...], sl_ref[...]) + dot(ql_ref[...], sh_ref[...]))

    qspec = pl.BlockSpec((bq, C), lambda i: (i, 0))
    sspec = pl.BlockSpec((C, HW), lambda i: (0, 0))
    return pl.pallas_call(
        body,
        grid=(nbq,),
        in_specs=[qspec, qspec, sspec, sspec],
        out_specs=pl.BlockSpec((bq, HW), lambda i: (i, 0)),
        out_shape=jax.ShapeDtypeStruct((hw, HW), jnp.float32),
    )(qh, ql, sh, sl)


_CQ = 2


def _lane_reduce(v, op):
    vals = [v[i] for i in range(16)]
    while len(vals) > 1:
        vals = [op(vals[i], vals[i + 1]) for i in range(0, len(vals) - 1, 2)] + (
            [vals[-1]] if len(vals) % 2 else [])
    return vals[0]


def _attend_sc(a_flat, rt, px, py, b, W, HW, C, K):
    CP = rt.shape[1]
    NQ = px.shape[0] // K
    NW = 32
    QPW = NQ // NW
    NC16 = C // 16
    NCH = QPW // _CQ
    NPAIR = NCH // 2
    mesh = plsc.VectorSubcoreMesh(core_axis_name="c", subcore_axis_name="s")

    @functools.partial(
        pl.kernel,
        out_type=[jax.ShapeDtypeStruct((NQ * K,), jnp.float32),
                  jax.ShapeDtypeStruct((NQ * C,), jnp.float32)],
        mesh=mesh,
        compiler_params=pltpu.CompilerParams(use_tc_tiling_on_sc=False),
        scratch_types=[
            pltpu.VMEM((QPW * K,), jnp.int32),
            pltpu.VMEM((QPW * K,), jnp.int32),
            pltpu.VMEM((QPW * K,), jnp.int32),
            pltpu.VMEM((QPW * K,), jnp.int32),
            pltpu.VMEM((QPW * K,), jnp.float32),
            pltpu.VMEM((QPW * K,), jnp.float32),
            pltpu.VMEM((_CQ * K, CP), jnp.float32),
            pltpu.VMEM((_CQ * K, CP), jnp.float32),
            pltpu.VMEM((QPW * C,), jnp.float32),
            pltpu.SemaphoreType.DMA,
            pltpu.SemaphoreType.DMA,
            pltpu.SemaphoreType.DMA,
        ],
    )
    def attend(a_hbm, rt_hbm, px_hbm, py_hbm, w_hbm, o_hbm,
               pxa, pya, sidx, vidx, sca, wa, vr0, vr1, oa,
               sem_s, sem_v0, sem_v1):
        cid = lax.axis_index("c")
        sid = lax.axis_index("s")
        wid = sid * 2 + cid
        qbase = wid * QPW
        vbase = b * HW
        SB = _CQ * K

        pltpu.sync_copy(px_hbm.at[pl.ds(qbase * K, QPW * K)], pxa)
        pltpu.sync_copy(py_hbm.at[pl.ds(qbase * K, QPW * K)], pya)

        def s_copy(t):
            return pltpu.make_async_copy(
                a_hbm.at[sidx.at[pl.ds(t * SB, SB)]],
                sca.at[pl.ds(t * SB, SB)], sem_s)

        vrow = (vr0, vr1)
        vsem = (sem_v0, sem_v1)

        def v_copy(t, buf):
            return pltpu.make_async_copy(
                rt_hbm.at[vidx.at[pl.ds(t * SB, SB)]], vrow[buf], vsem[buf])

        def build(t, carry):
            for r in range(_CQ):
                q = t * _CQ + r
                for j in range(K // 16):
                    o = q * K + j * 16
                    pf = pxa[pl.ds(o, 16)] * W + pya[pl.ds(o, 16)]
                    sidx[pl.ds(o, 16)] = (qbase + q) * HW + pf
                    vidx[pl.ds(o, 16)] = vbase + pf
            s_copy(t).start()
            return carry

        lax.fori_loop(0, NCH, build, 0)

        def drain_s(t, carry):
            s_copy(t).wait()
            return carry

        lax.fori_loop(0, NCH, drain_s, 0)

        def smax(t, carry):
            for r in range(_CQ):
                q = t * _CQ + r
                s0 = sca[pl.ds(q * K, 16)]
                s1 = sca[pl.ds(q * K + 16, 16)]
                vm = jnp.maximum(s0, s1)
                m = _lane_reduce(vm, jnp.maximum)
                e0 = jnp.exp(s0 - m)
                e1 = jnp.exp(s1 - m)
                den = _lane_reduce(e0 + e1, lax.add)
                wa[pl.ds(q * K, 16)] = e0 / den
                wa[pl.ds(q * K + 16, 16)] = e1 / den
            return carry

        lax.fori_loop(0, NCH, smax, 0)

        def combine(t, vr):
            for rr in range(_CQ):
                q = t * _CQ + rr
                acc = [jnp.zeros((16,), jnp.float32) for _ in range(NC16)]
                for j in range(K // 16):
                    wv = wa[pl.ds(q * K + j * 16, 16)]
                    for ii in range(16):
                        ws = wv[ii]
                        row = rr * K + j * 16 + ii
                        for cc in range(NC16):
                            acc[cc] = acc[cc] + ws * vr[row, pl.ds(cc * 16, 16)]
                for cc in range(NC16):
                    oa[pl.ds(q * C + cc * 16, 16)] = acc[cc]

        v_copy(0, 0).start()

        def pipe(tt, carry):
            for half in range(2):
                t = tt * 2 + half
                v_copy(t + 1, 1 - half).start()
                v_copy(t, half).wait()
                combine(t, vrow[half])
            return carry

        lax.fori_loop(0, NPAIR, pipe, 0)
        v_copy(NCH - 1, 0).wait()
        combine(NCH - 1, vrow[0])

        pltpu.sync_copy(wa, w_hbm.at[pl.ds(qbase * K, QPW * K)])
        pltpu.sync_copy(oa, o_hbm.at[pl.ds(qbase * C, QPW * C)])

    return attend(a_flat, rt, px, py)


def kernel(Q, S, R, Pos):
    B, C, h, w = Q.shape
    H, W = S.shape[2], S.shape[3]
    HW = H * W
    hw = h * w
    K = Pos.shape[-1]

    qmat = Q.reshape(B, C, hw).transpose(0, 2, 1)
    qh = qmat.astype(jnp.bfloat16)
    ql = (qmat - qh.astype(jnp.float32)).astype(jnp.bfloat16)
    sflat = S.reshape(B, C, HW)
    sh = sflat.astype(jnp.bfloat16)
    sl = (sflat - sh.astype(jnp.float32)).astype(jnp.bfloat16)

    rt = R.reshape(B, C, HW).transpose(0, 2, 1).reshape(B * HW, C)
    pxb = Pos[:, 0].reshape(B, hw * K)
    pyb = Pos[:, 1].reshape(B, hw * K)

    wts, orows = [], []
    for b in range(B):
        a = _scores_tc(qh[b], ql[b], sh[b], sl[b])
        wb, ob = _attend_sc(a.reshape(hw * HW), rt, pxb[b], pyb[b],
                            b, W, HW, C, K)
        wts.append(wb.reshape(hw, 1, K))
        orows.append(ob.reshape(h, w, C))

    M = jnp.stack(wts)
    buf = jnp.stack(orows).transpose(0, 3, 1, 2)
    return (buf, M)


# speedup vs baseline: 23.3010x; 1.0025x over previous
import functools

import jax
import jax.numpy as jnp
from jax import lax
from jax.experimental import pallas as pl
from jax.experimental.pallas import tpu as pltpu
from jax.experimental.pallas import tpu_sc as plsc


def _scores_tc(qh, ql, sh, sl, bq=392):
    hw, C = qh.shape
    HW = sh.shape[1]
    nbq = hw // bq

    def body(qh_ref, ql_ref, sh_ref, sl_ref, a_ref):
        dn = (((1,), (0,)), ((), ()))

        def dot(x, y):
            return lax.dot_general(x, y, dn,
                                   preferred_element_type=jnp.float32)

        a_ref[...] = dot(qh_ref[...], sh_ref[...]) + (
            dot(qh_ref[...], sl_ref[...]) + dot(ql_ref[...], sh_ref[...]))

    qspec = pl.BlockSpec((bq, C), lambda i: (i, 0))
    sspec = pl.BlockSpec((C, HW), lambda i: (0, 0))
    return pl.pallas_call(
        body,
        grid=(nbq,),
        in_specs=[qspec, qspec, sspec, sspec],
        out_specs=pl.BlockSpec((bq, HW), lambda i: (i, 0)),
        out_shape=jax.ShapeDtypeStruct((hw, HW), jnp.float32),
    )(qh, ql, sh, sl)


_CQ = 2


def _lane_reduce(v, op):
    vals = [v[i] for i in range(16)]
    while len(vals) > 1:
        vals = [op(vals[i], vals[i + 1]) for i in range(0, len(vals) - 1, 2)] + (
            [vals[-1]] if len(vals) % 2 else [])
    return vals[0]


def _attend_sc(a_flat, rt, px, py, b, W, HW, C, K):
    CP = rt.shape[1]
    NQ = px.shape[0] // K
    NW = 32
    QPW = NQ // NW
    NC16 = C // 16
    NCH = QPW // _CQ
    NPAIR = NCH // 2
    mesh = plsc.VectorSubcoreMesh(core_axis_name="c", subcore_axis_name="s")

    @functools.partial(
        pl.kernel,
        out_type=[jax.ShapeDtypeStruct((NQ * K,), jnp.float32),
                  jax.ShapeDtypeStruct((NQ * C,), jnp.float32)],
        mesh=mesh,
        compiler_params=pltpu.CompilerParams(use_tc_tiling_on_sc=False),
        scratch_types=[
            pltpu.VMEM((QPW * K,), jnp.int32),
            pltpu.VMEM((QPW * K,), jnp.int32),
            pltpu.VMEM((QPW * K,), jnp.int32),
            pltpu.VMEM((QPW * K,), jnp.int32),
            pltpu.VMEM((QPW * K,), jnp.float32),
            pltpu.VMEM((QPW * K,), jnp.float32),
            pltpu.VMEM((_CQ * K, CP), jnp.float32),
            pltpu.VMEM((_CQ * K, CP), jnp.float32),
            pltpu.VMEM((QPW * C,), jnp.float32),
            pltpu.SemaphoreType.DMA,
            pltpu.SemaphoreType.DMA,
            pltpu.SemaphoreType.DMA,
        ],
    )
    def attend(a_hbm, rt_hbm, px_hbm, py_hbm, w_hbm, o_hbm,
               pxa, pya, sidx, vidx, sca, wa, vr0, vr1, oa,
               sem_s, sem_v0, sem_v1):
        cid = lax.axis_index("c")
        sid = lax.axis_index("s")
        wid = sid * 2 + cid
        qbase = wid * QPW
        vbase = b * HW
        SB = _CQ * K

        pltpu.sync_copy(px_hbm.at[pl.ds(qbase * K, QPW * K)], pxa)
        pltpu.sync_copy(py_hbm.at[pl.ds(qbase * K, QPW * K)], pya)

        def s_copy(t):
            return pltpu.make_async_copy(
                a_hbm.at[sidx.at[pl.ds(t * SB, SB)]],
                sca.at[pl.ds(t * SB, SB)], sem_s)

        vrow = (vr0, vr1)
        vsem = (sem_v0, sem_v1)

        def v_copy(t, buf):
            return pltpu.make_async_copy(
                rt_hbm.at[vidx.at[pl.ds(t * SB, SB)]], vrow[buf], vsem[buf])

        def build(t, carry):
            for r in range(_CQ):
                q = t * _CQ + r
                for j in range(K // 16):
                    o = q * K + j * 16
                    pf = pxa[pl.ds(o, 16)] * W + pya[pl.ds(o, 16)]
                    sidx[pl.ds(o, 16)] = (qbase + q) * HW + pf
                    vidx[pl.ds(o, 16)] = vbase + pf
            s_copy(t).start()
            return carry

        lax.fori_loop(0, NCH, build, 0)

        def drain_s(t, carry):
            s_copy(t).wait()
            return carry

        lax.fori_loop(0, NCH, drain_s, 0)

        def smax(t, carry):
            for r in range(_CQ):
                q = t * _CQ + r
                s0 = sca[pl.ds(q * K, 16)]
                s1 = sca[pl.ds(q * K + 16, 16)]
                vm = jnp.maximum(s0, s1)
                m = _lane_reduce(vm, jnp.maximum)
                e0 = jnp.exp(s0 - m)
                e1 = jnp.exp(s1 - m)
                den = _lane_reduce(e0 + e1, lax.add)
                wa[pl.ds(q * K, 16)] = e0 / den
                wa[pl.ds(q * K + 16, 16)] = e1 / den
            return carry

        lax.fori_loop(0, NCH, smax, 0)

        def combine(t, vr):
            for rr in range(_CQ):
                q = t * _CQ + rr
                acc = [jnp.zeros((16,), jnp.float32) for _ in range(NC16)]
                for j in range(K // 16):
                    wv = wa[pl.ds(q * K + j * 16, 16)]
                    for ii in range(16):
                        ws = wv[ii]
                        row = rr * K + j * 16 + ii
                        for cc in range(NC16):
                            acc[cc] = acc[cc] + ws * vr[row, pl.ds(cc * 16, 16)]
                for cc in range(NC16):
                    oa[pl.ds(q * C + cc * 16, 16)] = acc[cc]

        v_copy(0, 0).start()

        def pipe(tt, carry):
            for half in range(2):
                t = tt * 2 + half
                v_copy(t + 1, 1 - half).start()
                v_copy(t, half).wait()
                combine(t, vrow[half])
            return carry

        lax.fori_loop(0, NPAIR, pipe, 0)
        v_copy(NCH - 1, 0).wait()
        combine(NCH - 1, vrow[0])

        pltpu.sync_copy(wa, w_hbm.at[pl.ds(qbase * K, QPW * K)])
        pltpu.sync_copy(oa, o_hbm.at[pl.ds(qbase * C, QPW * C)])

    return attend(a_flat, rt, px, py)


def kernel(Q, S, R, Pos):
    B, C, h, w = Q.shape
    H, W = S.shape[2], S.shape[3]
    HW = H * W
    hw = h * w
    K = Pos.shape[-1]

    qmat = Q.reshape(B, C, hw).transpose(0, 2, 1)
    qh = qmat.astype(jnp.bfloat16)
    ql = (qmat - qh.astype(jnp.float32)).astype(jnp.bfloat16)
    sflat = S.reshape(B, C, HW)
    sh = sflat.astype(jnp.bfloat16)
    sl = (sflat - sh.astype(jnp.float32)).astype(jnp.bfloat16)

    rt = R.reshape(B, C, HW).transpose(0, 2, 1).reshape(B * HW, C)
    pxb = Pos[:, 0].reshape(B, hw * K)
    pyb = Pos[:, 1].reshape(B, hw * K)

    wts, orows = [], []
    for b in range(B):
        a = _scores_tc(qh[b], ql[b], sh[b], sl[b])
        wb, ob = _attend_sc(a.reshape(hw * HW), rt, pxb[b], pyb[b],
                            b, W, HW, C, K)
        wts.append(wb.reshape(hw, 1, K))
        orows.append(ob.reshape(h, w, C).transpose(2, 0, 1))

    M = jnp.stack(wts)
    buf = jnp.stack(orows)
    return (buf, M)


# speedup vs baseline: 23.3748x; 1.0032x over previous
import functools

import jax
import jax.numpy as jnp
from jax import lax
from jax.experimental import pallas as pl
from jax.experimental.pallas import tpu as pltpu
from jax.experimental.pallas import tpu_sc as plsc


def _scores_tc(qh, ql, sh, sl, bq=392):
    hw, C = qh.shape
    HW = sh.shape[1]
    nbq = hw // bq

    def body(qh_ref, ql_ref, sh_ref, sl_ref, a_ref):
        dn = (((1,), (0,)), ((), ()))

        def dot(x, y):
            return lax.dot_general(x, y, dn,
                                   preferred_element_type=jnp.float32)

        a_ref[...] = dot(qh_ref[...], sh_ref[...]) + (
            dot(qh_ref[...], sl_ref[...]) + dot(ql_ref[...], sh_ref[...]))

    qspec = pl.BlockSpec((bq, C), lambda i: (i, 0))
    sspec = pl.BlockSpec((C, HW), lambda i: (0, 0))
    return pl.pallas_call(
        body,
        grid=(nbq,),
        in_specs=[qspec, qspec, sspec, sspec],
        out_specs=pl.BlockSpec((bq, HW), lambda i: (i, 0)),
        out_shape=jax.ShapeDtypeStruct((hw, HW), jnp.float32),
    )(qh, ql, sh, sl)


_CQ = 2


def _lane_reduce(v, op):
    vals = [v[i] for i in range(16)]
    while len(vals) > 1:
        vals = [op(vals[i], vals[i + 1]) for i in range(0, len(vals) - 1, 2)] + (
            [vals[-1]] if len(vals) % 2 else [])
    return vals[0]


def _attend_sc(a_flat, rt, px, py, b, W, HW, C, K):
    CP = rt.shape[1]
    NQ = px.shape[0] // K
    NW = 32
    QPW = NQ // NW
    NC16 = C // 16
    NCH = QPW // _CQ
    NPAIR = NCH // 2
    mesh = plsc.VectorSubcoreMesh(core_axis_name="c", subcore_axis_name="s")

    @functools.partial(
        pl.kernel,
        out_type=[jax.ShapeDtypeStruct((NQ * K,), jnp.float32),
                  jax.ShapeDtypeStruct((NQ * C,), jnp.float32)],
        mesh=mesh,
        compiler_params=pltpu.CompilerParams(use_tc_tiling_on_sc=False),
        scratch_types=[
            pltpu.VMEM((QPW * K,), jnp.int32),
            pltpu.VMEM((QPW * K,), jnp.int32),
            pltpu.VMEM((QPW * K,), jnp.int32),
            pltpu.VMEM((QPW * K,), jnp.int32),
            pltpu.VMEM((QPW * K,), jnp.float32),
            pltpu.VMEM((QPW * K,), jnp.float32),
            pltpu.VMEM((_CQ * K, CP), jnp.float32),
            pltpu.VMEM((_CQ * K, CP), jnp.float32),
            pltpu.VMEM((QPW * C,), jnp.float32),
            pltpu.SemaphoreType.DMA,
            pltpu.SemaphoreType.DMA,
            pltpu.SemaphoreType.DMA,
        ],
    )
    def attend(a_hbm, rt_hbm, px_hbm, py_hbm, w_hbm, o_hbm,
               pxa, pya, sidx, vidx, sca, wa, vr0, vr1, oa,
               sem_s, sem_v0, sem_v1):
        cid = lax.axis_index("c")
        sid = lax.axis_index("s")
        wid = sid * 2 + cid
        qbase = wid * QPW
        vbase = b * HW
        SB = _CQ * K

        with jax.named_scope("stage_pos"):
            pltpu.sync_copy(px_hbm.at[pl.ds(qbase * K, QPW * K)], pxa)
            pltpu.sync_copy(py_hbm.at[pl.ds(qbase * K, QPW * K)], pya)

        def s_copy(t):
            return pltpu.make_async_copy(
                a_hbm.at[sidx.at[pl.ds(t * SB, SB)]],
                sca.at[pl.ds(t * SB, SB)], sem_s)

        vrow = (vr0, vr1)
        vsem = (sem_v0, sem_v1)

        def v_copy(t, buf):
            return pltpu.make_async_copy(
                rt_hbm.at[vidx.at[pl.ds(t * SB, SB)]], vrow[buf], vsem[buf])

        def build(t, carry):
            for r in range(_CQ):
                q = t * _CQ + r
                for j in range(K // 16):
                    o = q * K + j * 16
                    pf = pxa[pl.ds(o, 16)] * W + pya[pl.ds(o, 16)]
                    sidx[pl.ds(o, 16)] = (qbase + q) * HW + pf
                    vidx[pl.ds(o, 16)] = vbase + pf
            s_copy(t).start()
            return carry

        with jax.named_scope("build_fire"):
            lax.fori_loop(0, NCH, build, 0)

        def drain_s(t, carry):
            s_copy(t).wait()
            return carry

        with jax.named_scope("drain_scores"):
            lax.fori_loop(0, NCH, drain_s, 0)

        def smax(t, carry):
            for r in range(_CQ):
                q = t * _CQ + r
                s0 = sca[pl.ds(q * K, 16)]
                s1 = sca[pl.ds(q * K + 16, 16)]
                vm = jnp.maximum(s0, s1)
                m = _lane_reduce(vm, jnp.maximum)
                e0 = jnp.exp(s0 - m)
                e1 = jnp.exp(s1 - m)
                den = _lane_reduce(e0 + e1, lax.add)
                wa[pl.ds(q * K, 16)] = e0 / den
                wa[pl.ds(q * K + 16, 16)] = e1 / den
            return carry

        with jax.named_scope("softmax"):
            lax.fori_loop(0, NCH, smax, 0)

        def combine(t, vr):
            for rr in range(_CQ):
                q = t * _CQ + rr
                acc = [jnp.zeros((16,), jnp.float32) for _ in range(NC16)]
                for j in range(K // 16):
                    wv = wa[pl.ds(q * K + j * 16, 16)]
                    for ii in range(16):
                        ws = wv[ii]
                        row = rr * K + j * 16 + ii
                        for cc in range(NC16):
                            acc[cc] = acc[cc] + ws * vr[row, pl.ds(cc * 16, 16)]
                for cc in range(NC16):
                    oa[pl.ds(q * C + cc * 16, 16)] = acc[cc]

        v_copy(0, 0).start()

        def pipe(tt, carry):
            for half in range(2):
                t = tt * 2 + half
                v_copy(t + 1, 1 - half).start()
                v_copy(t, half).wait()
                combine(t, vrow[half])
            return carry

        with jax.named_scope("value_pipe"):
            lax.fori_loop(0, NPAIR, pipe, 0)
            v_copy(NCH - 1, 0).wait()
            combine(NCH - 1, vrow[0])

        with jax.named_scope("writeback"):
            pltpu.sync_copy(wa, w_hbm.at[pl.ds(qbase * K, QPW * K)])
            pltpu.sync_copy(oa, o_hbm.at[pl.ds(qbase * C, QPW * C)])

    return attend(a_flat, rt, px, py)


def kernel(Q, S, R, Pos):
    B, C, h, w = Q.shape
    H, W = S.shape[2], S.shape[3]
    HW = H * W
    hw = h * w
    K = Pos.shape[-1]

    qmat = Q.reshape(B, C, hw).transpose(0, 2, 1)
    qh = qmat.astype(jnp.bfloat16)
    ql = (qmat - qh.astype(jnp.float32)).astype(jnp.bfloat16)
    sflat = S.reshape(B, C, HW)
    sh = sflat.astype(jnp.bfloat16)
    sl = (sflat - sh.astype(jnp.float32)).astype(jnp.bfloat16)

    rt = R.reshape(B, C, HW).transpose(0, 2, 1).reshape(B * HW, C)
    pxb = Pos[:, 0].reshape(B, hw * K)
    pyb = Pos[:, 1].reshape(B, hw * K)

    wts, orows = [], []
    for b in range(B):
        a = _scores_tc(qh[b], ql[b], sh[b], sl[b])
        wb, ob = _attend_sc(a.reshape(hw * HW), rt, pxb[b], pyb[b],
                            b, W, HW, C, K)
        wts.append(wb.reshape(hw, 1, K))
        orows.append(ob.reshape(h, w, C).transpose(2, 0, 1))

    M = jnp.stack(wts)
    buf = jnp.stack(orows)
    return (buf, M)
